# Initial kernel scaffold; baseline (speedup 1.0000x reference)
import jax
import jax.numpy as jnp
from jax.experimental import pallas as pl


def kernel(x, Wr, W1, W2):
    raise NotImplementedError("write your pallas kernel here")


# speedup vs baseline: 1.3408x; 1.3408x over previous
import functools

import jax
import jax.numpy as jnp
from jax.experimental import pallas as pl
from jax.experimental.pallas import tpu as pltpu


def _moe_body(x_ref, wr_ref, w1_ref, w2_ref, out_ref, scores_ref):
    e = pl.program_id(1)
    x = x_ref[...]

    @pl.when(e == 0)
    def _():
        logits = jax.lax.dot_general(
            x, wr_ref[...], (((1,), (1,)), ((), ())),
            preferred_element_type=jnp.float32)
        m = jnp.max(logits, axis=-1, keepdims=True)
        p = jnp.exp(logits - m)
        scores_ref[...] = p / jnp.sum(p, axis=-1, keepdims=True)
        out_ref[...] = jnp.zeros_like(out_ref)

    xb = x.astype(jnp.bfloat16)
    h = jax.lax.dot_general(
        xb, w1_ref[0], (((1,), (1,)), ((), ())),
        preferred_element_type=jnp.float32)
    h = h * jax.nn.sigmoid(h)
    scores = scores_ref[...]
    lane = jax.lax.broadcasted_iota(jnp.int32, scores.shape, 1)
    s = jnp.sum(jnp.where(lane == e, scores, 0.0), axis=-1, keepdims=True)
    hb = (h * s).astype(jnp.bfloat16)
    out_ref[...] += jax.lax.dot_general(
        hb, w2_ref[0], (((1,), (1,)), ((), ())),
        preferred_element_type=jnp.float32)


@functools.partial(jax.jit, static_argnames=())
def kernel(x, Wr, W1, W2):
    B, S, H = x.shape
    E, I, _ = W1.shape
    T = B * S
    xf = x.reshape(T, H)
    w1b = W1.astype(jnp.bfloat16)
    w2b = W2.astype(jnp.bfloat16)
    TM = 1024 if T % 1024 == 0 else T

    out = pl.pallas_call(
        _moe_body,
        grid=(T // TM, E),
        in_specs=[
            pl.BlockSpec((TM, H), lambda i, e: (i, 0)),
            pl.BlockSpec((E, H), lambda i, e: (0, 0)),
            pl.BlockSpec((1, I, H), lambda i, e: (e, 0, 0)),
            pl.BlockSpec((1, H, I), lambda i, e: (e, 0, 0)),
        ],
        out_specs=pl.BlockSpec((TM, H), lambda i, e: (i, 0)),
        out_shape=jax.ShapeDtypeStruct((T, H), jnp.float32),
        scratch_shapes=[pltpu.VMEM((TM, E), jnp.float32)],
        interpret=False,
    )(xf, Wr, w1b, w2b)
    return out.reshape(B, S, H)
